# Initial kernel scaffold; baseline (speedup 1.0000x reference)
import jax
import jax.numpy as jnp
from jax.experimental import pallas as pl


def kernel(indices, embedding, W, b):
    raise NotImplementedError("write your pallas kernel here")


# speedup vs baseline: 4.5662x; 4.5662x over previous
import functools

import jax
import jax.numpy as jnp
from jax import lax
from jax.experimental import pallas as pl
from jax.experimental.pallas import tpu as pltpu
from jax.experimental.pallas import tpu_sc as plsc

_NC = 2
_NS = 16
_NW = _NC * _NS
_CHUNK = 2048
_LANES = 16


def _project_body(emb_ref, w_ref, b_ref, out_ref):
    out_ref[...] = (
        lax.dot_general(
            emb_ref[...], w_ref[...], (((1,), (1,)), ((), ())),
            preferred_element_type=jnp.float32,
        )
        + b_ref[...]
    )


def _project_table(embedding, W, b):
    return pl.pallas_call(
        _project_body,
        out_shape=jax.ShapeDtypeStruct(embedding.shape, jnp.float32),
    )(embedding, W, b.reshape(1, -1))


def _make_gather(n_rows, d, table_words):
    b_per_w = n_rows // _NW
    n_chunks = b_per_w // _CHUNK
    assert b_per_w % _CHUNK == 0

    mesh = plsc.VectorSubcoreMesh(
        core_axis_name="c", subcore_axis_name="s",
        num_cores=_NC, num_subcores=_NS,
    )

    @functools.partial(
        pl.kernel,
        mesh=mesh,
        out_type=jax.ShapeDtypeStruct((n_rows * d,), jnp.float32),
        scratch_types=[
            pltpu.VMEM((table_words,), jnp.float32),
            pltpu.VMEM((_CHUNK,), jnp.int32),
            pltpu.VMEM((_CHUNK * d,), jnp.float32),
        ],
        compiler_params=pltpu.CompilerParams(needs_layout_passes=False),
    )
    def gather_kernel(table_hbm, idx_hbm, out_hbm, table_v, idx_v, rows_v):
        wid = lax.axis_index("s") * _NC + lax.axis_index("c")
        base = wid * b_per_w
        pltpu.sync_copy(table_hbm, table_v)
        lane = lax.iota(jnp.int32, _LANES)

        def chunk(g, carry):
            ib = base + g * _CHUNK
            pltpu.sync_copy(idx_hbm.at[pl.ds(ib, _CHUNK)], idx_v)

            def group(i, c):
                iv = idx_v[pl.ds(i * _LANES, _LANES)]
                src = iv * d
                dst = i * (_LANES * d) + lane * d
                for j in range(d):
                    vals = plsc.load_gather(table_v, [src + j])
                    plsc.store_scatter(rows_v, [dst + j], vals)
                return c

            lax.fori_loop(0, _CHUNK // _LANES, group, 0)
            pltpu.sync_copy(rows_v, out_hbm.at[pl.ds(ib * d, _CHUNK * d)])
            return carry

        lax.fori_loop(0, n_chunks, chunk, 0)

    return gather_kernel


def kernel(indices, embedding, W, b):
    bsz, seq = indices.shape
    d = embedding.shape[1]
    table = _project_table(embedding, W, b)
    flat = table.reshape(-1)
    table_words = (flat.shape[0] + 127) // 128 * 128
    flat = jnp.pad(flat, (0, table_words - flat.shape[0]))
    idx_flat = indices.reshape(-1).astype(jnp.int32)
    out = _make_gather(idx_flat.shape[0], d, table_words)(flat, idx_flat)
    return out.reshape(bsz, seq, d)


# speedup vs baseline: 5.4152x; 1.1859x over previous
import functools

import jax
import jax.numpy as jnp
from jax import lax
from jax.experimental import pallas as pl
from jax.experimental.pallas import tpu as pltpu
from jax.experimental.pallas import tpu_sc as plsc

_NC = 2
_NS = 16
_NW = _NC * _NS
_CHUNK = 2048
_LANES = 16


def _project_body(emb_ref, w_ref, b_ref, out_ref):
    out_ref[...] = (
        lax.dot_general(
            emb_ref[...], w_ref[...], (((1,), (1,)), ((), ())),
            preferred_element_type=jnp.float32,
        )
        + b_ref[...]
    )


def _project_table(embedding, W, b):
    return pl.pallas_call(
        _project_body,
        out_shape=jax.ShapeDtypeStruct(embedding.shape, jnp.float32),
    )(embedding, W, b.reshape(1, -1))


def _make_gather(n_rows, d, table_words):
    b_per_w = n_rows // _NW
    n_chunks = b_per_w // _CHUNK
    assert b_per_w % _CHUNK == 0

    mesh = plsc.VectorSubcoreMesh(
        core_axis_name="c", subcore_axis_name="s",
        num_cores=_NC, num_subcores=_NS,
    )

    @functools.partial(
        pl.kernel,
        mesh=mesh,
        out_type=jax.ShapeDtypeStruct((n_rows * d,), jnp.float32),
        scratch_types=[
            pltpu.VMEM((table_words,), jnp.float32),
            pltpu.VMEM((_CHUNK,), jnp.int32),
            pltpu.VMEM((_CHUNK * d,), jnp.float32),
        ],
        compiler_params=pltpu.CompilerParams(needs_layout_passes=False),
    )
    def gather_kernel(table_hbm, idx_hbm, out_hbm, table_v, idx_v, rows_v):
        wid = lax.axis_index("s") * _NC + lax.axis_index("c")
        base = wid * b_per_w
        pltpu.sync_copy(table_hbm, table_v)
        unroll = _LANES

        def chunk(g, carry):
            ib = base + g * _CHUNK
            pltpu.sync_copy(idx_hbm.at[pl.ds(ib, _CHUNK)], idx_v)

            def group(i, c):
                r0 = i * unroll
                iv = idx_v[pl.ds(r0, _LANES)]
                for k in range(unroll):
                    s = iv[k]
                    rows_v[pl.ds((r0 + k) * d, d)] = table_v[pl.ds(s * d, d)]
                return c

            lax.fori_loop(0, _CHUNK // unroll, group, 0)
            pltpu.sync_copy(rows_v, out_hbm.at[pl.ds(ib * d, _CHUNK * d)])
            return carry

        lax.fori_loop(0, n_chunks, chunk, 0)

    return gather_kernel


def kernel(indices, embedding, W, b):
    bsz, seq = indices.shape
    d = embedding.shape[1]
    table = _project_table(embedding, W, b)
    flat = table.reshape(-1)
    table_words = (flat.shape[0] + 127) // 128 * 128
    flat = jnp.pad(flat, (0, table_words - flat.shape[0]))
    idx_flat = indices.reshape(-1).astype(jnp.int32)
    out = _make_gather(idx_flat.shape[0], d, table_words)(flat, idx_flat)
    return out.reshape(bsz, seq, d)


# speedup vs baseline: 5.4326x; 1.0032x over previous
import functools

import jax
import jax.numpy as jnp
from jax import lax
from jax.experimental import pallas as pl
from jax.experimental.pallas import tpu as pltpu
from jax.experimental.pallas import tpu_sc as plsc

_NC = 2
_NS = 16
_NW = _NC * _NS
_CHUNK = 2048
_LANES = 16


def _project_body(emb_ref, w_ref, b_ref, out_ref):
    out_ref[...] = (
        lax.dot_general(
            emb_ref[...], w_ref[...], (((1,), (1,)), ((), ())),
            preferred_element_type=jnp.float32,
        )
        + b_ref[...]
    )


def _project_table(embedding, W, b):
    return pl.pallas_call(
        _project_body,
        out_shape=jax.ShapeDtypeStruct(embedding.shape, jnp.float32),
    )(embedding, W, b.reshape(1, -1))


def _make_gather(out_shape, d, table_words):
    n_rows = out_shape[0] * out_shape[1]
    b_per_w = n_rows // _NW
    n_chunks = b_per_w // _CHUNK
    assert b_per_w % _CHUNK == 0

    mesh = plsc.VectorSubcoreMesh(
        core_axis_name="c", subcore_axis_name="s",
        num_cores=_NC, num_subcores=_NS,
    )

    @functools.partial(
        pl.kernel,
        mesh=mesh,
        out_type=jax.ShapeDtypeStruct((n_rows, d), jnp.float32),
        scratch_types=[
            pltpu.VMEM((table_words,), jnp.float32),
            pltpu.VMEM((_CHUNK,), jnp.int32),
            pltpu.VMEM((_CHUNK, d), jnp.float32),
        ],
        compiler_params=pltpu.CompilerParams(
            needs_layout_passes=False, use_tc_tiling_on_sc=False,
        ),
    )
    def gather_kernel(table_hbm, idx_hbm, out_hbm, table_v, idx_v, rows_v):
        wid = lax.axis_index("s") * _NC + lax.axis_index("c")
        base = wid * b_per_w
        pltpu.sync_copy(table_hbm, table_v)
        unroll = _LANES

        def chunk(g, carry):
            ib = base + g * _CHUNK
            pltpu.sync_copy(idx_hbm.at[pl.ds(ib, _CHUNK)], idx_v)

            def group(i, c):
                r0 = i * unroll
                iv = idx_v[pl.ds(r0, _LANES)]
                for k in range(unroll):
                    s = iv[k]
                    rows_v[r0 + k, :] = table_v[pl.ds(s * d, d)]
                return c

            lax.fori_loop(0, _CHUNK // unroll, group, 0)
            pltpu.sync_copy(rows_v, out_hbm.at[pl.ds(ib, _CHUNK)])
            return carry

        lax.fori_loop(0, n_chunks, chunk, 0)

    return gather_kernel


def kernel(indices, embedding, W, b):
    bsz, seq = indices.shape
    d = embedding.shape[1]
    table = _project_table(embedding, W, b)
    flat = table.reshape(-1)
    table_words = (flat.shape[0] + 127) // 128 * 128
    flat = jnp.pad(flat, (0, table_words - flat.shape[0]))
    idx_flat = indices.reshape(-1).astype(jnp.int32)
    out = _make_gather((bsz, seq, d), d, table_words)(flat, idx_flat)
    return out.reshape(bsz, seq, d)
